# baseline (speedup 1.0000x reference)
import functools

import jax
import jax.numpy as jnp
from jax import lax
from jax.experimental import pallas as pl
from jax.experimental.pallas import tpu as pltpu
from jax.experimental.pallas import tpu_sc as plsc

ZCH = 1_000_000
EMBED_DIM = 64
TOTAL = 26 * 16384

L = 16
NC = 2
NS = 16
NW = NC * NS

CPW = TOTAL // NW
CH = 128
NCH = CPW // CH

SL = 62720
SPAD = NS * SL
ZC = 3920

_mesh = plsc.VectorSubcoreMesh(
    core_axis_name="c", subcore_axis_name="s", num_cores=NC, num_subcores=NS
)


@functools.partial(
    pl.kernel,
    out_type=(
        jax.ShapeDtypeStruct((TOTAL // CH, CH), jnp.int32),
        jax.ShapeDtypeStruct((NC, SPAD), jnp.float32),
    ),
    mesh=_mesh,
    compiler_params=pltpu.CompilerParams(use_tc_tiling_on_sc=False),
    scratch_types=[
        pltpu.VMEM((NCH, CH), jnp.int32),
        pltpu.VMEM((CH,), jnp.float32),
        pltpu.VMEM((ZC,), jnp.float32),
        pltpu.VMEM_SHARED((SPAD,), jnp.float32),
    ],
)
def _sc_remap_counts(values2d, remap, partials, idx2d, ones_v, zeros_v, counts_sp):
    cid = lax.axis_index("c")
    sid = lax.axis_index("s")
    wid = cid * NS + sid
    rbase = wid * NCH

    pltpu.sync_copy(values2d.at[pl.ds(rbase, NCH)], idx2d)

    def _rem_row(q, _):
        def _rem_vec(t, _):
            v = idx2d[q, pl.ds(t * L, L)]
            v = jnp.where(v >= 2 * ZCH, v - 2 * ZCH, v)
            v = jnp.where(v >= ZCH, v - ZCH, v)
            idx2d[q, pl.ds(t * L, L)] = v
            return 0
        return lax.fori_loop(0, CH // L, _rem_vec, 0, unroll=True)

    lax.fori_loop(0, NCH, _rem_row, 0)
    pltpu.sync_copy(idx2d, remap.at[pl.ds(rbase, NCH)])

    def _fill(t, _):
        ones_v[pl.ds(t * L, L)] = jnp.full((L,), 1.0, jnp.float32)
        return 0
    lax.fori_loop(0, CH // L, _fill, 0, unroll=True)

    def _zfill(t, _):
        zeros_v[pl.ds(t * L, L)] = jnp.zeros((L,), jnp.float32)
        return 0
    lax.fori_loop(0, ZC // L, _zfill, 0)

    def _zdma(t, _):
        pltpu.sync_copy(zeros_v, counts_sp.at[pl.ds(sid * SL + t * ZC, ZC)])
        return 0
    lax.fori_loop(0, SL // ZC, _zdma, 0)
    plsc.subcore_barrier()

    def _hist(j, _):
        pltpu.sync_copy(ones_v, counts_sp.at[idx2d.at[j]], add=True)
        return 0
    lax.fori_loop(0, NCH, _hist, 0)

    plsc.subcore_barrier()
    pltpu.sync_copy(
        counts_sp.at[pl.ds(sid * SL, SL)], partials.at[cid, pl.ds(sid * SL, SL)]
    )


NCH2 = NCH // 2
CPW2 = CPW // 2


def _make_sc_gather(half):
    @functools.partial(
        pl.kernel,
        out_type=jax.ShapeDtypeStruct((TOTAL // 2, 128), jnp.float32),
        mesh=_mesh,
        compiler_params=pltpu.CompilerParams(use_tc_tiling_on_sc=False),
        scratch_types=[
            pltpu.VMEM((NCH2, CH), jnp.int32),
            pltpu.VMEM((2, CH, 128), jnp.float32),
            pltpu.SemaphoreType.DMA,
            pltpu.SemaphoreType.DMA,
            pltpu.SemaphoreType.DMA,
        ],
        name=f"sc_gather_h{half}",
    )
    def _sc_gather(remap2d, table, emb, idx2d, rows, gsemA, gsemB, esem):
        cid = lax.axis_index("c")
        sid = lax.axis_index("s")
        wid = cid * NS + sid
        idrow = wid * NCH + half * NCH2
        rbase = wid * NCH2

        pltpu.sync_copy(remap2d.at[pl.ds(idrow, NCH2)], idx2d)

        pltpu.async_copy(table.at[idx2d.at[0]], rows.at[0], gsemA)

        def _pair(g, _):
            j0 = 2 * g
            j1 = j0 + 1

            @pl.when(g >= 1)
            def _():
                pltpu.make_async_copy(
                    rows.at[1], emb.at[pl.ds((rbase + j0 - 1) * CH, CH)], esem
                ).wait()

            pltpu.async_copy(table.at[idx2d.at[j1]], rows.at[1], gsemB)
            pltpu.make_async_copy(table.at[idx2d.at[j0]], rows.at[0], gsemA).wait()
            pltpu.async_copy(rows.at[0], emb.at[pl.ds((rbase + j0) * CH, CH)], esem)
            pltpu.make_async_copy(
                rows.at[0], emb.at[pl.ds((rbase + j0) * CH, CH)], esem
            ).wait()

            @pl.when(g + 1 < NCH2 // 2)
            def _():
                pltpu.async_copy(table.at[idx2d.at[j0 + 2]], rows.at[0], gsemA)

            pltpu.make_async_copy(table.at[idx2d.at[j1]], rows.at[1], gsemB).wait()
            pltpu.async_copy(rows.at[1], emb.at[pl.ds((rbase + j1) * CH, CH)], esem)
            return 0

        lax.fori_loop(0, NCH2 // 2, _pair, 0)
        pltpu.make_async_copy(
            rows.at[1], emb.at[pl.ds((rbase + NCH2 - 1) * CH, CH)], esem
        ).wait()

    return _sc_gather


_sc_gather_h = (_make_sc_gather(0), _make_sc_gather(1))


def _tr_in_body(t_ref, o_ref):
    o_ref[:, 0:64] = t_ref[...].T


_BKA = 8192
_NA = (ZCH + _BKA - 1) // _BKA


def _linearize_table(table):
    t64 = jnp.transpose(table)
    return pl.pallas_call(
        _tr_in_body,
        grid=(_NA,),
        in_specs=[pl.BlockSpec((64, _BKA), lambda j: (0, j))],
        out_specs=pl.BlockSpec((_BKA, 128), lambda j: (j, 0)),
        out_shape=jax.ShapeDtypeStruct((ZCH, 128), jnp.float32),
    )(t64)


def _tr_out_body(e_ref, o_ref):
    o_ref[...] = e_ref[:, 0:64].T


def _tr_out_body2(e_ref, _p_ref, o_ref):
    o_ref[...] = e_ref[:, 0:64].T


def _transpose_emb_half(emb_h, half, prev=None):
    out_spec = pl.BlockSpec((64, CPW2), lambda j, h=half: (0, 2 * j + h))
    out_shape = jax.ShapeDtypeStruct((64, TOTAL), jnp.float32)
    in_spec = pl.BlockSpec((CPW2, 128), lambda j: (j, 0))
    if prev is None:
        return pl.pallas_call(
            _tr_out_body, grid=(NW,), in_specs=[in_spec],
            out_specs=out_spec, out_shape=out_shape,
        )(emb_h)
    return pl.pallas_call(
        _tr_out_body2, grid=(NW,),
        in_specs=[in_spec, pl.BlockSpec((8, 128), lambda j: (0, 0))],
        out_specs=out_spec, out_shape=out_shape,
        input_output_aliases={1: 0},
    )(emb_h, prev)


def _combine_body(p_ref, o_ref):
    o_ref[...] = p_ref[0] + p_ref[1]


_BR = 784


def _combine(partials):
    p3 = partials.reshape(NC, SPAD // 128, 128)
    out = pl.pallas_call(
        _combine_body,
        grid=(SPAD // (128 * _BR),),
        in_specs=[pl.BlockSpec((NC, _BR, 128), lambda i: (0, i, 0))],
        out_specs=pl.BlockSpec((_BR, 128), lambda i: (i, 0)),
        out_shape=jax.ShapeDtypeStruct((SPAD // 128, 128), jnp.float32),
    )(p3)
    return out.reshape(SPAD)[:ZCH]


@jax.jit
def kernel(values, lengths, table):
    del lengths
    values2d = values.reshape(TOTAL // CH, CH)
    remap2d, partials = _sc_remap_counts(values2d)
    tpad = _linearize_table(table)
    emb_h0 = _sc_gather_h[0](remap2d, tpad)
    embt = _transpose_emb_half(emb_h0, 0)
    emb_h1 = _sc_gather_h[1](remap2d, tpad)
    embt = _transpose_emb_half(emb_h1, 1, embt)
    counts = _combine(partials)
    emb = jnp.transpose(embt)
    return emb, remap2d.reshape(TOTAL), counts


# speedup vs baseline: 1.0376x; 1.0376x over previous
import functools

import jax
import jax.numpy as jnp
from jax import lax
from jax.experimental import pallas as pl
from jax.experimental.pallas import tpu as pltpu
from jax.experimental.pallas import tpu_sc as plsc

ZCH = 1_000_000
EMBED_DIM = 64
TOTAL = 26 * 16384

L = 16
NC = 2
NS = 16
NW = NC * NS

CPW = TOTAL // NW
CH = 128
NCH = CPW // CH

SL = 62720
SPAD = NS * SL
ZC = 3920

_mesh = plsc.VectorSubcoreMesh(
    core_axis_name="c", subcore_axis_name="s", num_cores=NC, num_subcores=NS
)


@functools.partial(
    pl.kernel,
    out_type=(
        jax.ShapeDtypeStruct((TOTAL // CH, CH), jnp.int32),
        jax.ShapeDtypeStruct((NC, SPAD), jnp.float32),
    ),
    mesh=_mesh,
    compiler_params=pltpu.CompilerParams(use_tc_tiling_on_sc=False),
    scratch_types=[
        pltpu.VMEM((NCH, CH), jnp.int32),
        pltpu.VMEM((CH,), jnp.float32),
        pltpu.VMEM((ZC,), jnp.float32),
        pltpu.VMEM_SHARED((SPAD,), jnp.float32),
    ],
)
def _sc_remap_counts(values2d, remap, partials, idx2d, ones_v, zeros_v, counts_sp):
    cid = lax.axis_index("c")
    sid = lax.axis_index("s")
    wid = cid * NS + sid
    rbase = wid * NCH

    pltpu.sync_copy(values2d.at[pl.ds(rbase, NCH)], idx2d)

    def _rem_row(q, _):
        def _rem_vec(t, _):
            v = idx2d[q, pl.ds(t * L, L)]
            v = jnp.where(v >= 2 * ZCH, v - 2 * ZCH, v)
            v = jnp.where(v >= ZCH, v - ZCH, v)
            idx2d[q, pl.ds(t * L, L)] = v
            return 0
        return lax.fori_loop(0, CH // L, _rem_vec, 0, unroll=True)

    lax.fori_loop(0, NCH, _rem_row, 0)
    pltpu.sync_copy(idx2d, remap.at[pl.ds(rbase, NCH)])

    def _fill(t, _):
        ones_v[pl.ds(t * L, L)] = jnp.full((L,), 1.0, jnp.float32)
        return 0
    lax.fori_loop(0, CH // L, _fill, 0, unroll=True)

    def _zfill(t, _):
        zeros_v[pl.ds(t * L, L)] = jnp.zeros((L,), jnp.float32)
        return 0
    lax.fori_loop(0, ZC // L, _zfill, 0)

    def _zdma(t, _):
        pltpu.sync_copy(zeros_v, counts_sp.at[pl.ds(sid * SL + t * ZC, ZC)])
        return 0
    lax.fori_loop(0, SL // ZC, _zdma, 0)
    plsc.subcore_barrier()

    def _hist(j, _):
        pltpu.sync_copy(ones_v, counts_sp.at[idx2d.at[j]], add=True)
        return 0
    lax.fori_loop(0, NCH, _hist, 0)

    plsc.subcore_barrier()
    pltpu.sync_copy(
        counts_sp.at[pl.ds(sid * SL, SL)], partials.at[cid, pl.ds(sid * SL, SL)]
    )


NCH2 = NCH // 2
CPW2 = CPW // 2


def _make_sc_gather(half):
    @functools.partial(
        pl.kernel,
        out_type=jax.ShapeDtypeStruct((TOTAL // 2, 128), jnp.float32),
        mesh=_mesh,
        compiler_params=pltpu.CompilerParams(use_tc_tiling_on_sc=False),
        scratch_types=[
            pltpu.VMEM((NCH2, CH), jnp.int32),
            pltpu.VMEM((2, CH, 128), jnp.float32),
            pltpu.SemaphoreType.DMA,
            pltpu.SemaphoreType.DMA,
            pltpu.SemaphoreType.DMA,
        ],
        name=f"sc_gather_h{half}",
    )
    def _sc_gather(remap2d, table, emb, idx2d, rows, gsemA, gsemB, esem):
        cid = lax.axis_index("c")
        sid = lax.axis_index("s")
        wid = cid * NS + sid
        idrow = wid * NCH + half * NCH2
        rbase = wid * NCH2

        pltpu.sync_copy(remap2d.at[pl.ds(idrow, NCH2)], idx2d)

        pltpu.async_copy(table.at[idx2d.at[0]], rows.at[0], gsemA)

        def _pair(g, _):
            j0 = 2 * g
            j1 = j0 + 1

            @pl.when(g >= 1)
            def _():
                pltpu.make_async_copy(
                    rows.at[1], emb.at[pl.ds((rbase + j0 - 1) * CH, CH)], esem
                ).wait()

            pltpu.async_copy(table.at[idx2d.at[j1]], rows.at[1], gsemB)
            pltpu.make_async_copy(table.at[idx2d.at[j0]], rows.at[0], gsemA).wait()
            pltpu.async_copy(rows.at[0], emb.at[pl.ds((rbase + j0) * CH, CH)], esem)
            pltpu.make_async_copy(
                rows.at[0], emb.at[pl.ds((rbase + j0) * CH, CH)], esem
            ).wait()

            @pl.when(g + 1 < NCH2 // 2)
            def _():
                pltpu.async_copy(table.at[idx2d.at[j0 + 2]], rows.at[0], gsemA)

            pltpu.make_async_copy(table.at[idx2d.at[j1]], rows.at[1], gsemB).wait()
            pltpu.async_copy(rows.at[1], emb.at[pl.ds((rbase + j1) * CH, CH)], esem)
            return 0

        lax.fori_loop(0, NCH2 // 2, _pair, 0)
        pltpu.make_async_copy(
            rows.at[1], emb.at[pl.ds((rbase + NCH2 - 1) * CH, CH)], esem
        ).wait()

    return _sc_gather


_sc_gather_h = (_make_sc_gather(0), _make_sc_gather(1))


def _tr_in_body(t_ref, o_ref):
    o_ref[:, 0:64] = t_ref[...].T


_BKA = 16384
_NA = (ZCH + _BKA - 1) // _BKA


def _linearize_table(table):
    t64 = jnp.transpose(table)
    return pl.pallas_call(
        _tr_in_body,
        grid=(_NA,),
        in_specs=[pl.BlockSpec((64, _BKA), lambda j: (0, j))],
        out_specs=pl.BlockSpec((_BKA, 128), lambda j: (j, 0)),
        out_shape=jax.ShapeDtypeStruct((ZCH, 128), jnp.float32),
    )(t64)


def _tr_out_body(e_ref, o_ref):
    o_ref[...] = e_ref[:, 0:64].T


def _tr_out_body2(e_ref, _p_ref, o_ref):
    o_ref[...] = e_ref[:, 0:64].T


def _transpose_emb_half(emb_h, half, prev=None):
    out_spec = pl.BlockSpec((64, CPW2), lambda j, h=half: (0, 2 * j + h))
    out_shape = jax.ShapeDtypeStruct((64, TOTAL), jnp.float32)
    in_spec = pl.BlockSpec((CPW2, 128), lambda j: (j, 0))
    if prev is None:
        return pl.pallas_call(
            _tr_out_body, grid=(NW,), in_specs=[in_spec],
            out_specs=out_spec, out_shape=out_shape,
        )(emb_h)
    return pl.pallas_call(
        _tr_out_body2, grid=(NW,),
        in_specs=[in_spec, pl.BlockSpec((8, 128), lambda j: (0, 0))],
        out_specs=out_spec, out_shape=out_shape,
        input_output_aliases={1: 0},
    )(emb_h, prev)


def _combine_body(p_ref, o_ref):
    o_ref[...] = p_ref[0] + p_ref[1]


_BR = 784


def _combine(partials):
    p3 = partials.reshape(NC, SPAD // 128, 128)
    out = pl.pallas_call(
        _combine_body,
        grid=(SPAD // (128 * _BR),),
        in_specs=[pl.BlockSpec((NC, _BR, 128), lambda i: (0, i, 0))],
        out_specs=pl.BlockSpec((_BR, 128), lambda i: (i, 0)),
        out_shape=jax.ShapeDtypeStruct((SPAD // 128, 128), jnp.float32),
    )(p3)
    return out.reshape(SPAD)[:ZCH]


@jax.jit
def kernel(values, lengths, table):
    del lengths
    values2d = values.reshape(TOTAL // CH, CH)
    remap2d, partials = _sc_remap_counts(values2d)
    tpad = _linearize_table(table)
    emb_h0 = _sc_gather_h[0](remap2d, tpad)
    embt = _transpose_emb_half(emb_h0, 0)
    emb_h1 = _sc_gather_h[1](remap2d, tpad)
    embt = _transpose_emb_half(emb_h1, 1, embt)
    counts = _combine(partials)
    emb = jnp.transpose(embt)
    return emb, remap2d.reshape(TOTAL), counts


# speedup vs baseline: 1.0444x; 1.0065x over previous
import functools

import jax
import jax.numpy as jnp
from jax import lax
from jax.experimental import pallas as pl
from jax.experimental.pallas import tpu as pltpu
from jax.experimental.pallas import tpu_sc as plsc

ZCH = 1_000_000
EMBED_DIM = 64
TOTAL = 26 * 16384

L = 16
NC = 2
NS = 16
NW = NC * NS

CPW = TOTAL // NW
CH = 128
NCH = CPW // CH

SL = 62720
SPAD = NS * SL
ZC = 3920

_mesh = plsc.VectorSubcoreMesh(
    core_axis_name="c", subcore_axis_name="s", num_cores=NC, num_subcores=NS
)


@functools.partial(
    pl.kernel,
    out_type=(
        jax.ShapeDtypeStruct((TOTAL // CH, CH), jnp.int32),
        jax.ShapeDtypeStruct((NC, SPAD), jnp.float32),
    ),
    mesh=_mesh,
    compiler_params=pltpu.CompilerParams(use_tc_tiling_on_sc=False),
    scratch_types=[
        pltpu.VMEM((NCH, CH), jnp.int32),
        pltpu.VMEM((CH,), jnp.float32),
        pltpu.VMEM((ZC,), jnp.float32),
        pltpu.VMEM_SHARED((SPAD,), jnp.float32),
    ],
)
def _sc_remap_counts(values2d, remap, partials, idx2d, ones_v, zeros_v, counts_sp):
    cid = lax.axis_index("c")
    sid = lax.axis_index("s")
    wid = cid * NS + sid
    rbase = wid * NCH

    pltpu.sync_copy(values2d.at[pl.ds(rbase, NCH)], idx2d)

    def _rem_row(q, _):
        def _rem_vec(t, _):
            v = idx2d[q, pl.ds(t * L, L)]
            v = jnp.where(v >= 2 * ZCH, v - 2 * ZCH, v)
            v = jnp.where(v >= ZCH, v - ZCH, v)
            idx2d[q, pl.ds(t * L, L)] = v
            return 0
        return lax.fori_loop(0, CH // L, _rem_vec, 0, unroll=True)

    lax.fori_loop(0, NCH, _rem_row, 0)
    pltpu.sync_copy(idx2d, remap.at[pl.ds(rbase, NCH)])

    def _fill(t, _):
        ones_v[pl.ds(t * L, L)] = jnp.full((L,), 1.0, jnp.float32)
        return 0
    lax.fori_loop(0, CH // L, _fill, 0, unroll=True)

    def _zfill(t, _):
        zeros_v[pl.ds(t * L, L)] = jnp.zeros((L,), jnp.float32)
        return 0
    lax.fori_loop(0, ZC // L, _zfill, 0)

    def _zdma(t, _):
        pltpu.sync_copy(zeros_v, counts_sp.at[pl.ds(sid * SL + t * ZC, ZC)])
        return 0
    lax.fori_loop(0, SL // ZC, _zdma, 0)
    plsc.subcore_barrier()

    def _hist(j, _):
        pltpu.sync_copy(ones_v, counts_sp.at[idx2d.at[j]], add=True)
        return 0
    lax.fori_loop(0, NCH, _hist, 0)

    plsc.subcore_barrier()
    pltpu.sync_copy(
        counts_sp.at[pl.ds(sid * SL, SL)], partials.at[cid, pl.ds(sid * SL, SL)]
    )


NCH2 = NCH // 2
CPW2 = CPW // 2


def _make_sc_gather(half):
    @functools.partial(
        pl.kernel,
        out_type=jax.ShapeDtypeStruct((TOTAL // 2, 128), jnp.float32),
        mesh=_mesh,
        compiler_params=pltpu.CompilerParams(use_tc_tiling_on_sc=False),
        scratch_types=[
            pltpu.VMEM((NCH2, CH), jnp.int32),
            pltpu.VMEM((2, CH, 128), jnp.float32),
            pltpu.SemaphoreType.DMA,
            pltpu.SemaphoreType.DMA,
            pltpu.SemaphoreType.DMA,
        ],
        name=f"sc_gather_h{half}",
    )
    def _sc_gather(remap2d, table, emb, idx2d, rows, gsemA, gsemB, esem):
        cid = lax.axis_index("c")
        sid = lax.axis_index("s")
        wid = cid * NS + sid
        idrow = wid * NCH + half * NCH2
        rbase = wid * NCH2

        pltpu.sync_copy(remap2d.at[pl.ds(idrow, NCH2)], idx2d)

        pltpu.async_copy(table.at[idx2d.at[0]], rows.at[0], gsemA)

        def _pair(g, _):
            j0 = 2 * g
            j1 = j0 + 1

            @pl.when(g >= 1)
            def _():
                pltpu.make_async_copy(
                    rows.at[1], emb.at[pl.ds((rbase + j0 - 1) * CH, CH)], esem
                ).wait()

            pltpu.async_copy(table.at[idx2d.at[j1]], rows.at[1], gsemB)
            pltpu.make_async_copy(table.at[idx2d.at[j0]], rows.at[0], gsemA).wait()
            pltpu.async_copy(rows.at[0], emb.at[pl.ds((rbase + j0) * CH, CH)], esem)
            pltpu.make_async_copy(
                rows.at[0], emb.at[pl.ds((rbase + j0) * CH, CH)], esem
            ).wait()

            @pl.when(g + 1 < NCH2 // 2)
            def _():
                pltpu.async_copy(table.at[idx2d.at[j0 + 2]], rows.at[0], gsemA)

            pltpu.make_async_copy(table.at[idx2d.at[j1]], rows.at[1], gsemB).wait()
            pltpu.async_copy(rows.at[1], emb.at[pl.ds((rbase + j1) * CH, CH)], esem)
            return 0

        lax.fori_loop(0, NCH2 // 2, _pair, 0)
        pltpu.make_async_copy(
            rows.at[1], emb.at[pl.ds((rbase + NCH2 - 1) * CH, CH)], esem
        ).wait()

    return _sc_gather


_sc_gather_h = (_make_sc_gather(0), _make_sc_gather(1))


def _tr_in_body(t_ref, o_ref):
    o_ref[:, 0:64] = t_ref[...].T


_BKA = 25600
_NA = (ZCH + _BKA - 1) // _BKA


def _linearize_table(table):
    t64 = jnp.transpose(table)
    return pl.pallas_call(
        _tr_in_body,
        grid=(_NA,),
        in_specs=[pl.BlockSpec((64, _BKA), lambda j: (0, j))],
        out_specs=pl.BlockSpec((_BKA, 128), lambda j: (j, 0)),
        out_shape=jax.ShapeDtypeStruct((ZCH, 128), jnp.float32),
    )(t64)


def _tr_out_body(e_ref, o_ref):
    o_ref[...] = e_ref[:, 0:64].T


def _tr_out_body2(e_ref, _p_ref, o_ref):
    o_ref[...] = e_ref[:, 0:64].T


def _transpose_emb_half(emb_h, half, prev=None):
    out_spec = pl.BlockSpec((64, CPW2), lambda j, h=half: (0, 2 * j + h))
    out_shape = jax.ShapeDtypeStruct((64, TOTAL), jnp.float32)
    in_spec = pl.BlockSpec((CPW2, 128), lambda j: (j, 0))
    if prev is None:
        return pl.pallas_call(
            _tr_out_body, grid=(NW,), in_specs=[in_spec],
            out_specs=out_spec, out_shape=out_shape,
        )(emb_h)
    return pl.pallas_call(
        _tr_out_body2, grid=(NW,),
        in_specs=[in_spec, pl.BlockSpec((8, 128), lambda j: (0, 0))],
        out_specs=out_spec, out_shape=out_shape,
        input_output_aliases={1: 0},
    )(emb_h, prev)


def _combine_body(p_ref, o_ref):
    o_ref[...] = p_ref[0] + p_ref[1]


_BR = 784


def _combine(partials):
    p3 = partials.reshape(NC, SPAD // 128, 128)
    out = pl.pallas_call(
        _combine_body,
        grid=(SPAD // (128 * _BR),),
        in_specs=[pl.BlockSpec((NC, _BR, 128), lambda i: (0, i, 0))],
        out_specs=pl.BlockSpec((_BR, 128), lambda i: (i, 0)),
        out_shape=jax.ShapeDtypeStruct((SPAD // 128, 128), jnp.float32),
    )(p3)
    return out.reshape(SPAD)[:ZCH]


@jax.jit
def kernel(values, lengths, table):
    del lengths
    values2d = values.reshape(TOTAL // CH, CH)
    remap2d, partials = _sc_remap_counts(values2d)
    tpad = _linearize_table(table)
    emb_h0 = _sc_gather_h[0](remap2d, tpad)
    embt = _transpose_emb_half(emb_h0, 0)
    emb_h1 = _sc_gather_h[1](remap2d, tpad)
    embt = _transpose_emb_half(emb_h1, 1, embt)
    counts = _combine(partials)
    emb = jnp.transpose(embt)
    return emb, remap2d.reshape(TOTAL), counts
